# baseline (speedup 1.0000x reference)
import functools

import jax
import jax.numpy as jnp
from jax import lax
from jax.experimental import pallas as pl
from jax.experimental.pallas import tpu as pltpu
from jax.experimental.pallas import tpu_sc as plsc

_N_CLASS = 100000
_DIM = 16
_N_STEP = 20
_BATCH = 1024

_NW = 32
_ROWS_PER_W = (_N_STEP * _DIM) // _NW
_CHUNK = 128
_N_CHUNKS = _BATCH // _CHUNK


def _make_sc_gather():
    mesh = plsc.VectorSubcoreMesh(core_axis_name="c", subcore_axis_name="s")

    @functools.partial(
        pl.kernel,
        mesh=mesh,
        out_type=jax.ShapeDtypeStruct((_N_STEP * _DIM, _BATCH), jnp.float32),
        scratch_types=[
            pltpu.VMEM((_ROWS_PER_W, _BATCH), jnp.int32),
            pltpu.VMEM((_ROWS_PER_W, _BATCH), jnp.float32),
            pltpu.SemaphoreType.DMA,
            pltpu.SemaphoreType.DMA,
        ],
        compiler_params=pltpu.CompilerParams(use_tc_tiling_on_sc=False),
    )
    def gather_kernel(tflat_hbm, xt_hbm, out_hbm, xt_v, rows_v, sem, sem2):
        wid = lax.axis_index("s") * 2 + lax.axis_index("c")
        base = wid * _ROWS_PER_W
        loads = []
        for k in range(_ROWS_PER_W):
            t = (base + k) // _DIM
            loads.append(pltpu.async_copy(
                xt_hbm.at[pl.ds(t * _BATCH, _BATCH)], xt_v.at[k], sem2))
        for c in loads:
            c.wait()
        copies = []
        for k in range(_ROWS_PER_W):
            d = (base + k) % _DIM
            window = tflat_hbm.at[pl.ds(d * _N_CLASS, _N_CLASS)]
            for j in range(_N_CHUNKS):
                copies.append(pltpu.async_copy(
                    window.at[xt_v.at[k, pl.ds(j * _CHUNK, _CHUNK)]],
                    rows_v.at[k, pl.ds(j * _CHUNK, _CHUNK)],
                    sem,
                ))
        for c in copies:
            c.wait()
        pltpu.sync_copy(rows_v, out_hbm.at[pl.ds(base, _ROWS_PER_W)])

    return gather_kernel


_sc_gather = _make_sc_gather()


_BN = 6144
_BB = 512


def _proj_body(wt_ref, flat_ref, b_ref, out_ref):
    acc = lax.dot_general(
        wt_ref[...], flat_ref[...],
        (((0,), (0,)), ((), ())),
        preferred_element_type=jnp.float32,
    )
    bias_row = b_ref[...].reshape(1, _BN)
    ones_row = jnp.ones((1, _BB), jnp.float32)
    out_ref[...] = acc + lax.dot_general(
        bias_row, ones_row,
        (((0,), (0,)), ((), ())),
        preferred_element_type=jnp.float32,
    )


@jax.jit
def _projection(Wt, flat, b):
    nb = pl.cdiv(_N_CLASS, _BN)
    return pl.pallas_call(
        _proj_body,
        grid=(nb, _BATCH // _BB),
        in_specs=[
            pl.BlockSpec((_N_STEP * _DIM, _BN), lambda i, j: (0, i)),
            pl.BlockSpec((_N_STEP * _DIM, _BB), lambda i, j: (0, j)),
            pl.BlockSpec((_BN,), lambda i, j: (i,)),
        ],
        out_specs=pl.BlockSpec((_BN, _BB), lambda i, j: (i, j)),
        out_shape=jax.ShapeDtypeStruct((_N_CLASS, _BATCH), jnp.float32),
        compiler_params=pltpu.CompilerParams(
            dimension_semantics=("parallel", "arbitrary"),
        ),
    )(Wt, flat, b)


def kernel(x, table, W, b):
    xt = x.T.reshape(-1).astype(jnp.int32)
    tflat = table.T.reshape(-1)
    flatT = _sc_gather(tflat, xt)
    outT = _projection(W.T, flatT, b)
    return outT.T


# speedup vs baseline: 1.1495x; 1.1495x over previous
import functools

import jax
import jax.numpy as jnp
from jax import lax
from jax.experimental import pallas as pl
from jax.experimental.pallas import tpu as pltpu
from jax.experimental.pallas import tpu_sc as plsc

_N_CLASS = 100000
_DIM = 16
_N_STEP = 20
_BATCH = 1024

_NW = 32
_ROWS_PER_W = (_N_STEP * _DIM) // _NW
_CHUNK = 128
_N_CHUNKS = _BATCH // _CHUNK


def _make_sc_gather():
    mesh = plsc.VectorSubcoreMesh(core_axis_name="c", subcore_axis_name="s")

    @functools.partial(
        pl.kernel,
        mesh=mesh,
        out_type=jax.ShapeDtypeStruct((_N_STEP * _DIM, _BATCH), jnp.float32),
        scratch_types=[
            pltpu.VMEM((_ROWS_PER_W, _BATCH), jnp.int32),
            pltpu.VMEM((_ROWS_PER_W, _BATCH), jnp.float32),
            pltpu.SemaphoreType.DMA,
            pltpu.SemaphoreType.DMA,
        ],
        compiler_params=pltpu.CompilerParams(use_tc_tiling_on_sc=False),
    )
    def gather_kernel(tflat_hbm, xt_hbm, out_hbm, xt_v, rows_v, sem, sem2):
        wid = lax.axis_index("s") * 2 + lax.axis_index("c")
        base = wid * _ROWS_PER_W
        loads = []
        for k in range(_ROWS_PER_W):
            t = (base + k) // _DIM
            loads.append(pltpu.async_copy(
                xt_hbm.at[pl.ds(t * _BATCH, _BATCH)], xt_v.at[k], sem2))
        for c in loads:
            c.wait()
        copies = []
        for k in range(_ROWS_PER_W):
            d = (base + k) % _DIM
            window = tflat_hbm.at[pl.ds(d * _N_CLASS, _N_CLASS)]
            for j in range(_N_CHUNKS):
                copies.append(pltpu.async_copy(
                    window.at[xt_v.at[k, pl.ds(j * _CHUNK, _CHUNK)]],
                    rows_v.at[k, pl.ds(j * _CHUNK, _CHUNK)],
                    sem,
                ))
        for c in copies:
            c.wait()
        pltpu.sync_copy(rows_v, out_hbm.at[pl.ds(base, _ROWS_PER_W)])

    return gather_kernel


_sc_gather = _make_sc_gather()


_BN = 4096
_BB = 1024


def _proj_body(wt_ref, flat_ref, b_ref, out_ref):
    acc = lax.dot_general(
        wt_ref[...], flat_ref[...],
        (((0,), (0,)), ((), ())),
        preferred_element_type=jnp.float32,
    )
    bias_row = b_ref[...].reshape(1, _BN)
    ones_row = jnp.ones((1, _BB), jnp.float32)
    out_ref[...] = acc + lax.dot_general(
        bias_row, ones_row,
        (((0,), (0,)), ((), ())),
        preferred_element_type=jnp.float32,
    )


@jax.jit
def _projection(Wt, flat, b):
    nb = pl.cdiv(_N_CLASS, _BN)
    return pl.pallas_call(
        _proj_body,
        grid=(nb, _BATCH // _BB),
        in_specs=[
            pl.BlockSpec((_N_STEP * _DIM, _BN), lambda i, j: (0, i)),
            pl.BlockSpec((_N_STEP * _DIM, _BB), lambda i, j: (0, j)),
            pl.BlockSpec((_BN,), lambda i, j: (i,)),
        ],
        out_specs=pl.BlockSpec((_BN, _BB), lambda i, j: (i, j)),
        out_shape=jax.ShapeDtypeStruct((_N_CLASS, _BATCH), jnp.float32),
        compiler_params=pltpu.CompilerParams(
            dimension_semantics=("parallel", "arbitrary"),
        ),
    )(Wt, flat, b)


def kernel(x, table, W, b):
    xt = x.T.reshape(-1).astype(jnp.int32)
    tflat = table.T.reshape(-1)
    flatT = _sc_gather(tflat, xt)
    outT = _projection(W.T, flatT, b)
    return outT.T


# speedup vs baseline: 1.1583x; 1.0076x over previous
import functools

import jax
import jax.numpy as jnp
from jax import lax
from jax.experimental import pallas as pl
from jax.experimental.pallas import tpu as pltpu
from jax.experimental.pallas import tpu_sc as plsc

_N_CLASS = 100000
_DIM = 16
_N_STEP = 20
_BATCH = 1024

_NW = 32
_ROWS_PER_W = (_N_STEP * _DIM) // _NW
_CHUNK = 1024
_N_CHUNKS = _BATCH // _CHUNK


def _make_sc_gather():
    mesh = plsc.VectorSubcoreMesh(core_axis_name="c", subcore_axis_name="s")

    @functools.partial(
        pl.kernel,
        mesh=mesh,
        out_type=jax.ShapeDtypeStruct((_N_STEP * _DIM, _BATCH), jnp.float32),
        scratch_types=[
            pltpu.VMEM((2, _BATCH), jnp.int32),
            pltpu.VMEM((_ROWS_PER_W, _BATCH), jnp.float32),
            pltpu.SemaphoreType.DMA,
            pltpu.SemaphoreType.DMA,
        ],
        compiler_params=pltpu.CompilerParams(use_tc_tiling_on_sc=False),
    )
    def gather_kernel(tflat_hbm, xt_hbm, out_hbm, xt_v, rows_v, sem, sem2):
        wid = lax.axis_index("s") * 2 + lax.axis_index("c")
        base = wid * _ROWS_PER_W
        t0 = base // _DIM
        t1 = (base + _ROWS_PER_W - 1) // _DIM
        l0 = pltpu.async_copy(
            xt_hbm.at[pl.ds(t0 * _BATCH, _BATCH)], xt_v.at[0], sem2)
        l1 = pltpu.async_copy(
            xt_hbm.at[pl.ds(t1 * _BATCH, _BATCH)], xt_v.at[1], sem2)
        l0.wait()
        l1.wait()
        copies = []
        for k in range(_ROWS_PER_W):
            d = (base + k) % _DIM
            tk = (base + k) // _DIM - t0
            window = tflat_hbm.at[pl.ds(d * _N_CLASS, _N_CLASS)]
            for j in range(_N_CHUNKS):
                copies.append(pltpu.async_copy(
                    window.at[xt_v.at[tk, pl.ds(j * _CHUNK, _CHUNK)]],
                    rows_v.at[k, pl.ds(j * _CHUNK, _CHUNK)],
                    sem,
                ))
        for c in copies:
            c.wait()
        pltpu.sync_copy(rows_v, out_hbm.at[pl.ds(base, _ROWS_PER_W)])

    return gather_kernel


_sc_gather = _make_sc_gather()


_BN = 4096
_BB = 1024


def _proj_body(wt_ref, flat_ref, b_ref, out_ref):
    acc = lax.dot_general(
        wt_ref[...], flat_ref[...],
        (((0,), (0,)), ((), ())),
        preferred_element_type=jnp.float32,
    )
    bias_row = b_ref[...].reshape(1, _BN)
    ones_row = jnp.ones((1, _BB), jnp.float32)
    out_ref[...] = acc + lax.dot_general(
        bias_row, ones_row,
        (((0,), (0,)), ((), ())),
        preferred_element_type=jnp.float32,
    )


@jax.jit
def _projection(Wt, flat, b):
    nb = pl.cdiv(_N_CLASS, _BN)
    return pl.pallas_call(
        _proj_body,
        grid=(nb, _BATCH // _BB),
        in_specs=[
            pl.BlockSpec((_N_STEP * _DIM, _BN), lambda i, j: (0, i)),
            pl.BlockSpec((_N_STEP * _DIM, _BB), lambda i, j: (0, j)),
            pl.BlockSpec((_BN,), lambda i, j: (i,)),
        ],
        out_specs=pl.BlockSpec((_BN, _BB), lambda i, j: (i, j)),
        out_shape=jax.ShapeDtypeStruct((_N_CLASS, _BATCH), jnp.float32),
        compiler_params=pltpu.CompilerParams(
            dimension_semantics=("parallel", "arbitrary"),
        ),
    )(Wt, flat, b)


def kernel(x, table, W, b):
    xt = x.T.reshape(-1).astype(jnp.int32)
    tflat = table.T.reshape(-1)
    flatT = _sc_gather(tflat, xt)
    outT = _projection(W.T, flatT, b)
    return outT.T
